# baseline (speedup 1.0000x reference)
import functools

import jax
import jax.numpy as jnp
from jax import lax
from jax.experimental import pallas as pl
from jax.experimental.pallas import tpu as pltpu
from jax.experimental.pallas import tpu_sc as plsc

N = 10000
E = 160000
D = 64
DP = 128
MAX_N = 9
NBLOCKS = 4
N_ELEM = 10
CUTOFF = 5.0

NC = 2
NS = 16
NW = NC * NS

CHUNK = 128
E_PAD = 163840
EPW = E_PAD // NW
NCH = EPW // CHUNK
N_PAD = 10240
RPT = N_PAD // NS

EBLK = 4096


def _sc_mesh():
    return plsc.VectorSubcoreMesh(core_axis_name="c", subcore_axis_name="s",
                                  num_cores=NC, num_subcores=NS)


G = 4
NGRP = NCH // G


def _gather_body(h_hbm, sd_hbm, hs_hbm, hd_hbm,
                 idx_v, buf_a, buf_b, sem_a, sem_b):
    c = lax.axis_index("c")
    s = lax.axis_index("s")
    w = c * NS + s
    base = w * EPW

    pltpu.sync_copy(sd_hbm.at[w], idx_v)

    bufs = ((buf_a, sem_a), (buf_b, sem_b))

    def fire(ep, g, buf, sem):
        return [pltpu.async_copy(h_hbm.at[idx_v.at[ep, g * G + k]],
                                 buf.at[pl.ds(k * CHUNK, CHUNK)], sem)
                for k in range(G)]

    pend = {0: fire(0, 0, *bufs[0])}
    for ep, out_hbm in enumerate((hs_hbm, hd_hbm)):
        for g in range(NGRP):
            pslot = g % 2
            if g + 1 < NGRP:
                pend[(g + 1) % 2] = fire(ep, g + 1, *bufs[(g + 1) % 2])
            elif ep == 0:
                pend[(g + 1) % 2] = fire(1, 0, *bufs[(g + 1) % 2])
            for hnd in pend.pop(pslot):
                hnd.wait()
            pltpu.sync_copy(bufs[pslot][0],
                            out_hbm.at[pl.ds(base + g * G * CHUNK, G * CHUNK)])


def _sc_gather(h16, sd4):
    f = pl.kernel(
        _gather_body,
        out_type=(jax.ShapeDtypeStruct((E_PAD, DP), jnp.bfloat16),
                  jax.ShapeDtypeStruct((E_PAD, DP), jnp.bfloat16)),
        mesh=_sc_mesh(),
        scratch_types=[
            pltpu.VMEM((2, NCH, CHUNK), jnp.int32),
            pltpu.VMEM((G * CHUNK, DP), jnp.bfloat16),
            pltpu.VMEM((G * CHUNK, DP), jnp.bfloat16),
            pltpu.SemaphoreType.DMA,
            pltpu.SemaphoreType.DMA,
        ],
        compiler_params=pltpu.CompilerParams(use_tc_tiling_on_sc=False),
    )
    return f(h16, sd4)


GS = 1
NGRPS = NCH // GS


def _scatter_body(msg_hbm, dst_hbm, zero_hbm, agg_hbm,
                  idx_v, buf_a, buf_b, acc_sh, sem_a, sem_b):
    c = lax.axis_index("c")
    s = lax.axis_index("s")
    w = c * NS + s
    base = w * EPW

    pltpu.sync_copy(zero_hbm.at[pl.ds(s * RPT, RPT)],
                    acc_sh.at[pl.ds(s * RPT, RPT)])
    pltpu.sync_copy(dst_hbm.at[w], idx_v)
    plsc.subcore_barrier()

    bufs = ((buf_a, sem_a), (buf_b, sem_b))

    def fire(g, buf, sem):
        return pltpu.async_copy(
            msg_hbm.at[pl.ds(base + g * GS * CHUNK, GS * CHUNK)], buf, sem)

    pend = {0: fire(0, *bufs[0])}
    for g in range(NGRPS):
        pslot = g % 2
        if g + 1 < NGRPS:
            pend[(g + 1) % 2] = fire(g + 1, *bufs[(g + 1) % 2])
        pend.pop(pslot).wait()
        buf = bufs[pslot][0]
        for k in range(GS):
            pltpu.sync_copy(buf.at[pl.ds(k * CHUNK, CHUNK)],
                            acc_sh.at[idx_v.at[g * GS + k]], add=True)

    plsc.subcore_barrier()
    pltpu.sync_copy(acc_sh.at[pl.ds(s * RPT, RPT)],
                    agg_hbm.at[c, pl.ds(s * RPT, RPT)])


def _sc_scatter(msg, dst3, zeros):
    f = pl.kernel(
        _scatter_body,
        out_type=jax.ShapeDtypeStruct((NC, N_PAD, DP), jnp.float32),
        mesh=_sc_mesh(),
        scratch_types=[
            pltpu.VMEM((NCH, CHUNK), jnp.int32),
            pltpu.VMEM((GS * CHUNK, DP), jnp.float32),
            pltpu.VMEM((GS * CHUNK, DP), jnp.float32),
            pltpu.VMEM_SHARED((N_PAD, DP), jnp.float32),
            pltpu.SemaphoreType.DMA,
            pltpu.SemaphoreType.DMA,
        ],
        compiler_params=pltpu.CompilerParams(use_tc_tiling_on_sc=False),
    )
    return f(msg, dst3, zeros)


def _h0_body(nt_ref, emb_ref, out_ref, out16_ref):
    nt = nt_ref[:, 0]
    onehot = (nt[:, None] == lax.broadcasted_iota(jnp.int32, (1, N_ELEM), 1)
              ).astype(jnp.float32)
    h = jnp.dot(onehot, emb_ref[...], preferred_element_type=jnp.float32)
    out_ref[...] = h
    out16_ref[...] = h.astype(jnp.bfloat16)


def _tc_h0(nt2, embed_p):
    return pl.pallas_call(
        _h0_body,
        out_shape=(jax.ShapeDtypeStruct((N, DP), jnp.float32),
                   jax.ShapeDtypeStruct((N, DP), jnp.bfloat16)),
    )(nt2, embed_p)


def _edge_body(first, last, *refs):
    if first:
        (hs_ref, hd_ref, d_ref, wrbf_ref,
         w1a, w1b, w1c, wga, wgb, wgc, wea, web, wec) = refs[:13]
        d = d_ref[...]
        u = d * (1.0 / CUTOFF)
        x = jnp.pi * u
        s1 = jnp.sin(x)
        c2 = 2.0 * jnp.cos(x)
        env = jnp.where(u < 1.0, (1.0 - u ** 5) ** 2, 0.0)
        scale = env * ((2.0 / CUTOFF) ** 0.5 / d)
        sk_m1, sk = jnp.zeros_like(s1), s1
        e = (sk * scale) * wrbf_ref[0:1, :]
        for k in range(1, MAX_N):
            sk, sk_m1 = c2 * sk - sk_m1, sk
            e = e + (sk * scale) * wrbf_ref[k:k + 1, :]
    else:
        (hs_ref, hd_ref, e_ref,
         w1a, w1b, w1c, wga, wgb, wgc, wea, web, wec) = refs[:12]
        e = e_ref[...]
    outs = refs[13:] if first else refs[12:]
    hs = hs_ref[...]
    hd = hd_ref[...]
    e16 = e.astype(jnp.bfloat16)

    def three(wa, wb, wc):
        return (jnp.dot(hs, wa[...], preferred_element_type=jnp.float32)
                + jnp.dot(hd, wb[...], preferred_element_type=jnp.float32)
                + jnp.dot(e16, wc[...], preferred_element_type=jnp.float32))

    z1 = three(w1a, w1b, w1c)
    zg = three(wga, wgb, wgc)
    msg = z1 * jax.nn.sigmoid(z1) * jax.nn.sigmoid(zg)
    outs[0][...] = jnp.concatenate([msg, jnp.zeros_like(msg)], axis=1)
    if not last:
        ze = three(wea, web, wec)
        outs[1][...] = e + ze * jax.nn.sigmoid(ze)


def _tc_edge(hs, hd, e_or_d, wrbf, wparts, first, last):
    grid = (E_PAD // EBLK,)
    row = pl.BlockSpec((EBLK, e_or_d.shape[1]), lambda i: (i, 0))
    full = lambda a: pl.BlockSpec(a.shape, lambda i: (0, 0))
    rowP = pl.BlockSpec((EBLK, DP), lambda i: (i, 0))
    rowD = pl.BlockSpec((EBLK, D), lambda i: (i, 0))
    in_specs = [rowP, rowP, row]
    args = [hs, hd, e_or_d]
    if first:
        in_specs.append(full(wrbf))
        args.append(wrbf)
    for wp in wparts:
        in_specs.append(full(wp))
        args.append(wp)
    out_shape = [jax.ShapeDtypeStruct((E_PAD, DP), jnp.float32)]
    out_specs = [rowP]
    if not last:
        out_shape.append(jax.ShapeDtypeStruct((E_PAD, D), jnp.float32))
        out_specs.append(rowD)
    res = pl.pallas_call(
        functools.partial(_edge_body, first, last),
        grid=grid,
        in_specs=in_specs,
        out_specs=out_specs,
        out_shape=out_shape,
        compiler_params=pltpu.CompilerParams(
            dimension_semantics=("arbitrary",)),
    )(*args)
    return res if not last else (res[0], None)


def _update_body(h_ref, agg_ref, out_ref, out16_ref):
    hn = h_ref[...] + agg_ref[0, 0:N, :] + agg_ref[1, 0:N, :]
    out_ref[...] = hn
    out16_ref[...] = hn.astype(jnp.bfloat16)


def _tc_update(h, agg):
    return pl.pallas_call(
        _update_body,
        out_shape=(jax.ShapeDtypeStruct((N, DP), jnp.float32),
                   jax.ShapeDtypeStruct((N, DP), jnp.bfloat16)),
    )(h, agg)


def _readout_body(h_ref, agg_ref, wout_ref, out_ref):
    hn = h_ref[...] + agg_ref[0, 0:N, :] + agg_ref[1, 0:N, :]
    out_ref[...] = jnp.sum(hn * wout_ref[...], axis=1, keepdims=True)


def _tc_readout(h, agg, wout2):
    return pl.pallas_call(
        _readout_body,
        out_shape=jax.ShapeDtypeStruct((N, 1), jnp.float32),
    )(h, agg, wout2)


def kernel(node_types, edge_index, edge_dist, embed, W_rbf, W1, Wg, We, Wout):
    src = edge_index[0].astype(jnp.int32)
    dst = edge_index[1].astype(jnp.int32)
    pad = E_PAD - E
    src3 = jnp.concatenate([src, jnp.zeros((pad,), jnp.int32)]
                           ).reshape(NW, NCH, CHUNK)
    dst3 = jnp.concatenate([dst, jnp.full((pad,), N, jnp.int32)]
                           ).reshape(NW, NCH, CHUNK)
    sd4 = jnp.stack([src3, dst3], axis=1)
    d2 = jnp.concatenate([edge_dist, jnp.ones((pad,), jnp.float32)]
                         ).reshape(E_PAD, 1)
    zeros = jnp.zeros((N_PAD, DP), jnp.float32)
    nt2 = node_types.astype(jnp.int32).reshape(N, 1)
    embed_p = jnp.pad(embed, ((0, 0), (0, DP - D)))
    wout2 = jnp.pad(Wout, (0, DP - D)).reshape(1, DP)
    zpad = jnp.zeros((DP - D, D), jnp.float32)

    h, h16 = _tc_h0(nt2, embed_p)
    e = d2
    agg = None
    for b in range(NBLOCKS):
        wparts = []
        for W in (W1, Wg, We):
            wb = W[b]
            wparts.append(jnp.concatenate([wb[0:D], zpad]
                                          ).astype(jnp.bfloat16))
            wparts.append(jnp.concatenate([wb[D:2 * D], zpad]
                                          ).astype(jnp.bfloat16))
            wparts.append(wb[2 * D:3 * D].astype(jnp.bfloat16))
        wparts = [wparts[0], wparts[1], wparts[2],
                  wparts[3], wparts[4], wparts[5],
                  wparts[6], wparts[7], wparts[8]]
        hs, hd = _sc_gather(h16, sd4)
        first = b == 0
        last = b == NBLOCKS - 1
        msg, e = _tc_edge(hs, hd, e, W_rbf, wparts, first, last)
        agg = _sc_scatter(msg, dst3, zeros)
        if not last:
            h, h16 = _tc_update(h, agg)
    out = _tc_readout(h, agg, wout2)
    return out.reshape(N)


# speedup vs baseline: 1.5191x; 1.5191x over previous
import functools

import jax
import jax.numpy as jnp
from jax import lax
from jax.experimental import pallas as pl
from jax.experimental.pallas import tpu as pltpu
from jax.experimental.pallas import tpu_sc as plsc

N = 10000
E = 160000
D = 64
DP = 128
MAX_N = 9
NBLOCKS = 4
N_ELEM = 10
CUTOFF = 5.0

NC = 2
NS = 16
NW = NC * NS

CHUNK = 128
E_PAD = 163840
EPW = E_PAD // NW
NCH = EPW // CHUNK
N_PAD = 10240
RPT = N_PAD // NS

EBLK = 4096


def _sc_mesh():
    return plsc.VectorSubcoreMesh(core_axis_name="c", subcore_axis_name="s",
                                  num_cores=NC, num_subcores=NS)


G = 4
NGRP = NCH // G


def _gather_body(h_hbm, sd_hbm, hs_hbm, hd_hbm,
                 idx_v, buf_a, buf_b, sem_a, sem_b):
    c = lax.axis_index("c")
    s = lax.axis_index("s")
    w = c * NS + s
    base = w * EPW

    pltpu.sync_copy(sd_hbm.at[w], idx_v)

    bufs = ((buf_a, sem_a), (buf_b, sem_b))

    def fire(ep, g, buf, sem):
        return [pltpu.async_copy(h_hbm.at[idx_v.at[ep, g * G + k]],
                                 buf.at[pl.ds(k * CHUNK, CHUNK)], sem)
                for k in range(G)]

    pend = {0: fire(0, 0, *bufs[0])}
    for ep, out_hbm in enumerate((hs_hbm, hd_hbm)):
        for g in range(NGRP):
            pslot = g % 2
            if g + 1 < NGRP:
                pend[(g + 1) % 2] = fire(ep, g + 1, *bufs[(g + 1) % 2])
            elif ep == 0:
                pend[(g + 1) % 2] = fire(1, 0, *bufs[(g + 1) % 2])
            for hnd in pend.pop(pslot):
                hnd.wait()
            pltpu.sync_copy(bufs[pslot][0],
                            out_hbm.at[pl.ds(base + g * G * CHUNK, G * CHUNK)])


def _sc_gather(h16, sd4):
    f = pl.kernel(
        _gather_body,
        out_type=(jax.ShapeDtypeStruct((E_PAD, D), jnp.bfloat16),
                  jax.ShapeDtypeStruct((E_PAD, D), jnp.bfloat16)),
        mesh=_sc_mesh(),
        scratch_types=[
            pltpu.VMEM((2, NCH, CHUNK), jnp.int32),
            pltpu.VMEM((G * CHUNK, D), jnp.bfloat16),
            pltpu.VMEM((G * CHUNK, D), jnp.bfloat16),
            pltpu.SemaphoreType.DMA,
            pltpu.SemaphoreType.DMA,
        ],
        compiler_params=pltpu.CompilerParams(use_tc_tiling_on_sc=False),
    )
    return f(h16, sd4)


GS = 1
NGRPS = NCH // GS


def _scatter_body(msg_hbm, dst_hbm, zero_hbm, agg_hbm,
                  idx_v, buf_a, buf_b, acc_sh, sem_a, sem_b):
    c = lax.axis_index("c")
    s = lax.axis_index("s")
    w = c * NS + s
    base = w * EPW

    pltpu.sync_copy(zero_hbm.at[pl.ds(s * RPT, RPT)],
                    acc_sh.at[pl.ds(s * RPT, RPT)])
    pltpu.sync_copy(dst_hbm.at[w], idx_v)
    plsc.subcore_barrier()

    bufs = ((buf_a, sem_a), (buf_b, sem_b))

    def fire(g, buf, sem):
        return pltpu.async_copy(
            msg_hbm.at[pl.ds(base + g * GS * CHUNK, GS * CHUNK)], buf, sem)

    pend = {0: fire(0, *bufs[0])}
    for g in range(NGRPS):
        pslot = g % 2
        if g + 1 < NGRPS:
            pend[(g + 1) % 2] = fire(g + 1, *bufs[(g + 1) % 2])
        pend.pop(pslot).wait()
        buf = bufs[pslot][0]
        for k in range(GS):
            pltpu.sync_copy(buf.at[pl.ds(k * CHUNK, CHUNK)],
                            acc_sh.at[idx_v.at[g * GS + k]], add=True)

    plsc.subcore_barrier()
    pltpu.sync_copy(acc_sh.at[pl.ds(s * RPT, RPT)],
                    agg_hbm.at[c, pl.ds(s * RPT, RPT)])


def _sc_scatter(msg, dst3, zeros):
    f = pl.kernel(
        _scatter_body,
        out_type=jax.ShapeDtypeStruct((NC, N_PAD, DP), jnp.float32),
        mesh=_sc_mesh(),
        scratch_types=[
            pltpu.VMEM((NCH, CHUNK), jnp.int32),
            pltpu.VMEM((GS * CHUNK, DP), jnp.float32),
            pltpu.VMEM((GS * CHUNK, DP), jnp.float32),
            pltpu.VMEM_SHARED((N_PAD, DP), jnp.float32),
            pltpu.SemaphoreType.DMA,
            pltpu.SemaphoreType.DMA,
        ],
        compiler_params=pltpu.CompilerParams(use_tc_tiling_on_sc=True),
    )
    return f(msg, dst3, zeros)


def _h0_body(nt_ref, emb_ref, out_ref, out16_ref):
    nt = nt_ref[:, 0]
    onehot = (nt[:, None] == lax.broadcasted_iota(jnp.int32, (1, N_ELEM), 1)
              ).astype(jnp.float32)
    h = jnp.dot(onehot, emb_ref[...], preferred_element_type=jnp.float32)
    out_ref[...] = h
    out16_ref[...] = h.astype(jnp.bfloat16)


def _tc_h0(nt2, embed):
    return pl.pallas_call(
        _h0_body,
        out_shape=(jax.ShapeDtypeStruct((N, D), jnp.float32),
                   jax.ShapeDtypeStruct((N, D), jnp.bfloat16)),
    )(nt2, embed)


def _edge_body(first, last, *refs):
    if first:
        (hs_ref, hd_ref, d_ref, wrbf_ref,
         w1a, w1b, w1c, wga, wgb, wgc, wea, web, wec) = refs[:13]
        d = d_ref[...]
        u = d * (1.0 / CUTOFF)
        n = (lax.broadcasted_iota(jnp.int32, (1, MAX_N), 1) + 1
             ).astype(jnp.float32)
        sbf = jnp.sin(n * (jnp.pi * u)) * ((2.0 / CUTOFF) ** 0.5 / d)
        env = jnp.where(u < 1.0, (1.0 - u ** 5) ** 2, 0.0)
        e = jnp.dot(sbf * env, wrbf_ref[...],
                    preferred_element_type=jnp.float32)
    else:
        (hs_ref, hd_ref, e_ref,
         w1a, w1b, w1c, wga, wgb, wgc, wea, web, wec) = refs[:12]
        e = e_ref[...]
    outs = refs[13:] if first else refs[12:]
    hs = hs_ref[...]
    hd = hd_ref[...]
    e16 = e.astype(jnp.bfloat16)

    def three(wa, wb, wc):
        return (jnp.dot(hs, wa[...], preferred_element_type=jnp.float32)
                + jnp.dot(hd, wb[...], preferred_element_type=jnp.float32)
                + jnp.dot(e16, wc[...], preferred_element_type=jnp.float32))

    z1 = three(w1a, w1b, w1c)
    zg = three(wga, wgb, wgc)
    msg = z1 * jax.nn.sigmoid(z1) * jax.nn.sigmoid(zg)
    outs[0][...] = jnp.concatenate([msg, jnp.zeros_like(msg)], axis=1)
    if not last:
        ze = three(wea, web, wec)
        outs[1][...] = e + ze * jax.nn.sigmoid(ze)


def _tc_edge(hs, hd, e_or_d, wrbf, wparts, first, last):
    grid = (E_PAD // EBLK,)
    row = pl.BlockSpec((EBLK, e_or_d.shape[1]), lambda i: (i, 0))
    full = lambda a: pl.BlockSpec(a.shape, lambda i: (0, 0))
    rowP = pl.BlockSpec((EBLK, DP), lambda i: (i, 0))
    rowD = pl.BlockSpec((EBLK, D), lambda i: (i, 0))
    in_specs = [rowD, rowD, row]
    args = [hs, hd, e_or_d]
    if first:
        in_specs.append(full(wrbf))
        args.append(wrbf)
    for wp in wparts:
        in_specs.append(full(wp))
        args.append(wp)
    out_shape = [jax.ShapeDtypeStruct((E_PAD, DP), jnp.float32)]
    out_specs = [rowP]
    if not last:
        out_shape.append(jax.ShapeDtypeStruct((E_PAD, D), jnp.float32))
        out_specs.append(rowD)
    res = pl.pallas_call(
        functools.partial(_edge_body, first, last),
        grid=grid,
        in_specs=in_specs,
        out_specs=out_specs,
        out_shape=out_shape,
        compiler_params=pltpu.CompilerParams(
            dimension_semantics=("arbitrary",)),
    )(*args)
    return res if not last else (res[0], None)


def _update_body(h_ref, agg_ref, out_ref, out16_ref):
    hn = (h_ref[...] + agg_ref[0, 0:N, 0:D] + agg_ref[1, 0:N, 0:D])
    out_ref[...] = hn
    out16_ref[...] = hn.astype(jnp.bfloat16)


def _tc_update(h, agg):
    return pl.pallas_call(
        _update_body,
        out_shape=(jax.ShapeDtypeStruct((N, D), jnp.float32),
                   jax.ShapeDtypeStruct((N, D), jnp.bfloat16)),
    )(h, agg)


def _readout_body(h_ref, agg_ref, wout_ref, out_ref):
    hn = h_ref[...] + agg_ref[0, 0:N, 0:D] + agg_ref[1, 0:N, 0:D]
    out_ref[...] = jnp.sum(hn * wout_ref[...], axis=1, keepdims=True)


def _tc_readout(h, agg, wout2):
    return pl.pallas_call(
        _readout_body,
        out_shape=jax.ShapeDtypeStruct((N, 1), jnp.float32),
    )(h, agg, wout2)


def kernel(node_types, edge_index, edge_dist, embed, W_rbf, W1, Wg, We, Wout):
    src = edge_index[0].astype(jnp.int32)
    dst = edge_index[1].astype(jnp.int32)
    pad = E_PAD - E
    src3 = jnp.concatenate([src, jnp.zeros((pad,), jnp.int32)]
                           ).reshape(NW, NCH, CHUNK)
    dst3 = jnp.concatenate([dst, jnp.full((pad,), N, jnp.int32)]
                           ).reshape(NW, NCH, CHUNK)
    sd4 = jnp.stack([src3, dst3], axis=1)
    d2 = jnp.concatenate([edge_dist, jnp.ones((pad,), jnp.float32)]
                         ).reshape(E_PAD, 1)
    zeros = jnp.zeros((N_PAD, DP), jnp.float32)
    nt2 = node_types.astype(jnp.int32).reshape(N, 1)
    wout2 = Wout.reshape(1, D)

    h, h16 = _tc_h0(nt2, embed)
    e = d2
    agg = None
    for b in range(NBLOCKS):
        wparts = []
        for W in (W1, Wg, We):
            wb = W[b]
            wparts.extend([wb[0:D].astype(jnp.bfloat16),
                           wb[D:2 * D].astype(jnp.bfloat16),
                           wb[2 * D:3 * D].astype(jnp.bfloat16)])
        hs, hd = _sc_gather(h16, sd4)
        first = b == 0
        last = b == NBLOCKS - 1
        msg, e = _tc_edge(hs, hd, e, W_rbf, wparts, first, last)
        agg = _sc_scatter(msg, dst3, zeros)
        if not last:
            h, h16 = _tc_update(h, agg)
    out = _tc_readout(h, agg, wout2)
    return out.reshape(N)


# speedup vs baseline: 1.5401x; 1.0138x over previous
import functools

import jax
import jax.numpy as jnp
from jax import lax
from jax.experimental import pallas as pl
from jax.experimental.pallas import tpu as pltpu
from jax.experimental.pallas import tpu_sc as plsc

N = 10000
E = 160000
D = 64
DP = 128
MAX_N = 9
NBLOCKS = 4
N_ELEM = 10
CUTOFF = 5.0

NC = 2
NS = 16
NW = NC * NS

CHUNK = 128
E_PAD = 163840
NH = 2
E_H = E_PAD // NH
EPW = E_H // NW
NCH = EPW // CHUNK
N_PAD = 10240
RPT = N_PAD // NS

EBLK = 4096


def _sc_mesh():
    return plsc.VectorSubcoreMesh(core_axis_name="c", subcore_axis_name="s",
                                  num_cores=NC, num_subcores=NS)


G = 4
NGRP = NCH // G


def _gather_body(h_hbm, sd_hbm, hs_hbm, hd_hbm,
                 idx_v, buf_a, buf_b, sem_a, sem_b):
    c = lax.axis_index("c")
    s = lax.axis_index("s")
    w = c * NS + s
    base = w * EPW

    pltpu.sync_copy(sd_hbm.at[w], idx_v)

    bufs = ((buf_a, sem_a), (buf_b, sem_b))

    def fire(ep, g, buf, sem):
        return [pltpu.async_copy(h_hbm.at[idx_v.at[ep, g * G + k]],
                                 buf.at[pl.ds(k * CHUNK, CHUNK)], sem)
                for k in range(G)]

    outs = (hs_hbm, hd_hbm)
    tasks = [(ep, g) for ep in range(2) for g in range(NGRP)]
    pend = {0: fire(*tasks[0], *bufs[0])}
    for t, (ep, g) in enumerate(tasks):
        pslot = t % 2
        if t + 1 < len(tasks):
            pend[(t + 1) % 2] = fire(*tasks[t + 1], *bufs[(t + 1) % 2])
        for hnd in pend.pop(pslot):
            hnd.wait()
        pltpu.sync_copy(bufs[pslot][0],
                        outs[ep].at[pl.ds(base + g * G * CHUNK, G * CHUNK)])


def _sc_gather(h16, sd4):
    f = pl.kernel(
        _gather_body,
        out_type=(jax.ShapeDtypeStruct((E_H, D), jnp.bfloat16),
                  jax.ShapeDtypeStruct((E_H, D), jnp.bfloat16)),
        mesh=_sc_mesh(),
        scratch_types=[
            pltpu.VMEM((2, NCH, CHUNK), jnp.int32),
            pltpu.VMEM((G * CHUNK, D), jnp.bfloat16),
            pltpu.VMEM((G * CHUNK, D), jnp.bfloat16),
            pltpu.SemaphoreType.DMA,
            pltpu.SemaphoreType.DMA,
        ],
        compiler_params=pltpu.CompilerParams(use_tc_tiling_on_sc=False),
    )
    return f(h16, sd4)


GS = 1
NGRPS = NCH // GS


def _scatter_body(msg_hbm, dst_hbm, zero_hbm, agg_hbm,
                  idx_v, buf_a, buf_b, acc_sh, sem_a, sem_b):
    c = lax.axis_index("c")
    s = lax.axis_index("s")
    w = c * NS + s
    base = w * EPW

    pltpu.sync_copy(zero_hbm.at[pl.ds(s * RPT, RPT)],
                    acc_sh.at[pl.ds(s * RPT, RPT)])
    pltpu.sync_copy(dst_hbm.at[w], idx_v)
    plsc.subcore_barrier()

    bufs = ((buf_a, sem_a), (buf_b, sem_b))

    def fire(g, buf, sem):
        return pltpu.async_copy(
            msg_hbm.at[pl.ds(base + g * GS * CHUNK, GS * CHUNK)], buf, sem)

    pend = {0: fire(0, *bufs[0])}
    for g in range(NGRPS):
        pslot = g % 2
        if g + 1 < NGRPS:
            pend[(g + 1) % 2] = fire(g + 1, *bufs[(g + 1) % 2])
        pend.pop(pslot).wait()
        buf = bufs[pslot][0]
        for k in range(GS):
            pltpu.sync_copy(buf.at[pl.ds(k * CHUNK, CHUNK)],
                            acc_sh.at[idx_v.at[g * GS + k]], add=True)

    plsc.subcore_barrier()
    pltpu.sync_copy(acc_sh.at[pl.ds(s * RPT, RPT)],
                    agg_hbm.at[c, pl.ds(s * RPT, RPT)])


def _sc_scatter(msg, dst3, zeros):
    f = pl.kernel(
        _scatter_body,
        out_type=jax.ShapeDtypeStruct((NC, N_PAD, DP), jnp.float32),
        mesh=_sc_mesh(),
        scratch_types=[
            pltpu.VMEM((NCH, CHUNK), jnp.int32),
            pltpu.VMEM((GS * CHUNK, DP), jnp.float32),
            pltpu.VMEM((GS * CHUNK, DP), jnp.float32),
            pltpu.VMEM_SHARED((N_PAD, DP), jnp.float32),
            pltpu.SemaphoreType.DMA,
            pltpu.SemaphoreType.DMA,
        ],
        compiler_params=pltpu.CompilerParams(use_tc_tiling_on_sc=True),
    )
    return f(msg, dst3, zeros)


def _h0_body(nt_ref, emb_ref, out_ref, out16_ref):
    nt = nt_ref[:, 0]
    onehot = (nt[:, None] == lax.broadcasted_iota(jnp.int32, (1, N_ELEM), 1)
              ).astype(jnp.float32)
    h = jnp.dot(onehot, emb_ref[...], preferred_element_type=jnp.float32)
    out_ref[...] = h
    out16_ref[...] = h.astype(jnp.bfloat16)


def _tc_h0(nt2, embed):
    return pl.pallas_call(
        _h0_body,
        out_shape=(jax.ShapeDtypeStruct((N, D), jnp.float32),
                   jax.ShapeDtypeStruct((N, D), jnp.bfloat16)),
    )(nt2, embed)


def _edge_body(first, last, *refs):
    if first:
        (hs_ref, hd_ref, d_ref, wrbf_ref,
         w1a, w1b, w1c, wga, wgb, wgc, wea, web, wec) = refs[:13]
        d = d_ref[...]
        u = d * (1.0 / CUTOFF)
        n = (lax.broadcasted_iota(jnp.int32, (1, MAX_N), 1) + 1
             ).astype(jnp.float32)
        sbf = jnp.sin(n * (jnp.pi * u)) * ((2.0 / CUTOFF) ** 0.5 / d)
        env = jnp.where(u < 1.0, (1.0 - u ** 5) ** 2, 0.0)
        e = jnp.dot(sbf * env, wrbf_ref[...],
                    preferred_element_type=jnp.float32)
    else:
        (hs_ref, hd_ref, e_ref,
         w1a, w1b, w1c, wga, wgb, wgc, wea, web, wec) = refs[:12]
        e = e_ref[...]
    outs = refs[13:] if first else refs[12:]
    hs = hs_ref[...]
    hd = hd_ref[...]
    e16 = e.astype(jnp.bfloat16)

    def three(wa, wb, wc):
        return (jnp.dot(hs, wa[...], preferred_element_type=jnp.float32)
                + jnp.dot(hd, wb[...], preferred_element_type=jnp.float32)
                + jnp.dot(e16, wc[...], preferred_element_type=jnp.float32))

    z1 = three(w1a, w1b, w1c)
    zg = three(wga, wgb, wgc)
    msg = z1 * jax.nn.sigmoid(z1) * jax.nn.sigmoid(zg)
    outs[0][...] = jnp.concatenate([msg, jnp.zeros_like(msg)], axis=1)
    if not last:
        ze = three(wea, web, wec)
        outs[1][...] = e + ze * jax.nn.sigmoid(ze)


def _tc_edge(hs, hd, e_or_d, wrbf, wparts, first, last):
    grid = (E_H // EBLK,)
    row = pl.BlockSpec((EBLK, e_or_d.shape[1]), lambda i: (i, 0))
    full = lambda a: pl.BlockSpec(a.shape, lambda i: (0, 0))
    rowP = pl.BlockSpec((EBLK, DP), lambda i: (i, 0))
    rowD = pl.BlockSpec((EBLK, D), lambda i: (i, 0))
    in_specs = [rowD, rowD, row]
    args = [hs, hd, e_or_d]
    if first:
        in_specs.append(full(wrbf))
        args.append(wrbf)
    for wp in wparts:
        in_specs.append(full(wp))
        args.append(wp)
    out_shape = [jax.ShapeDtypeStruct((E_H, DP), jnp.float32)]
    out_specs = [rowP]
    if not last:
        out_shape.append(jax.ShapeDtypeStruct((E_H, D), jnp.float32))
        out_specs.append(rowD)
    res = pl.pallas_call(
        functools.partial(_edge_body, first, last),
        grid=grid,
        in_specs=in_specs,
        out_specs=out_specs,
        out_shape=out_shape,
        compiler_params=pltpu.CompilerParams(
            dimension_semantics=("arbitrary",)),
    )(*args)
    return res if not last else (res[0], None)


def _update_body(h_ref, agg_a_ref, agg_b_ref, out_ref, out16_ref):
    hn = (h_ref[...]
          + agg_a_ref[0, 0:N, 0:D] + agg_a_ref[1, 0:N, 0:D]
          + agg_b_ref[0, 0:N, 0:D] + agg_b_ref[1, 0:N, 0:D])
    out_ref[...] = hn
    out16_ref[...] = hn.astype(jnp.bfloat16)


def _tc_update(h, agg_a, agg_b):
    return pl.pallas_call(
        _update_body,
        out_shape=(jax.ShapeDtypeStruct((N, D), jnp.float32),
                   jax.ShapeDtypeStruct((N, D), jnp.bfloat16)),
    )(h, agg_a, agg_b)


def _readout_body(h_ref, agg_a_ref, agg_b_ref, wout_ref, out_ref):
    hn = (h_ref[...]
          + agg_a_ref[0, 0:N, 0:D] + agg_a_ref[1, 0:N, 0:D]
          + agg_b_ref[0, 0:N, 0:D] + agg_b_ref[1, 0:N, 0:D])
    out_ref[...] = jnp.sum(hn * wout_ref[...], axis=1, keepdims=True)


def _tc_readout(h, agg_a, agg_b, wout2):
    return pl.pallas_call(
        _readout_body,
        out_shape=jax.ShapeDtypeStruct((N, 1), jnp.float32),
    )(h, agg_a, agg_b, wout2)


def kernel(node_types, edge_index, edge_dist, embed, W_rbf, W1, Wg, We, Wout):
    src = edge_index[0].astype(jnp.int32)
    dst = edge_index[1].astype(jnp.int32)
    pad = E_PAD - E
    src_h = jnp.concatenate([src, jnp.zeros((pad,), jnp.int32)]
                            ).reshape(NH, NW, NCH, CHUNK)
    dst_h = jnp.concatenate([dst, jnp.full((pad,), N, jnp.int32)]
                            ).reshape(NH, NW, NCH, CHUNK)
    sd4 = [jnp.stack([src_h[i], dst_h[i]], axis=1) for i in range(NH)]
    d_all = jnp.concatenate([edge_dist, jnp.ones((pad,), jnp.float32)]
                            ).reshape(NH, E_H, 1)
    zeros = jnp.zeros((N_PAD, DP), jnp.float32)
    nt2 = node_types.astype(jnp.int32).reshape(N, 1)
    wout2 = Wout.reshape(1, D)

    h, h16 = _tc_h0(nt2, embed)
    e = [d_all[0], d_all[1]]
    agg = [None, None]
    for b in range(NBLOCKS):
        wparts = []
        for W in (W1, Wg, We):
            wb = W[b]
            wparts.extend([wb[0:D].astype(jnp.bfloat16),
                           wb[D:2 * D].astype(jnp.bfloat16),
                           wb[2 * D:3 * D].astype(jnp.bfloat16)])
        first = b == 0
        last = b == NBLOCKS - 1
        gath = [_sc_gather(h16, sd4[i]) for i in range(NH)]
        for i in range(NH):
            hs, hd = gath[i]
            msg, e_i = _tc_edge(hs, hd, e[i], W_rbf, wparts, first, last)
            agg[i] = _sc_scatter(msg, dst_h[i], zeros)
            e[i] = e_i
        if not last:
            h, h16 = _tc_update(h, agg[0], agg[1])
    out = _tc_readout(h, agg[0], agg[1], wout2)
    return out.reshape(N)
